# baseline (speedup 1.0000x reference)
import jax
import jax.numpy as jnp
from jax.experimental import pallas as pl
from jax.experimental.pallas import tpu as pltpu


def _dma_kernel(starts_ref, curr_ref, hist_ref, out_ref, sem_h, sem_c):
    B, H = hist_ref.shape[0], hist_ref.shape[1]
    hist_copies = []
    curr_copies = []
    for bs in range(B):
        c = pltpu.make_async_copy(
            hist_ref.at[bs, pl.ds(1, H - 1)],
            out_ref.at[pl.ds(bs * H + 1, H - 1)],
            sem_h.at[bs],
        )
        c.start()
        hist_copies.append(c)
    for bs in range(B):
        c = pltpu.make_async_copy(
            curr_ref.at[pl.ds(starts_ref[bs], 1)],
            out_ref.at[pl.ds(bs * H, 1)],
            sem_c.at[bs],
        )
        c.start()
        curr_copies.append(c)
    for c in hist_copies:
        c.wait()
    for c in curr_copies:
        c.wait()


def kernel(feat_curr, feat_history, record_len):
    B, H, C, Hh, W = feat_history.shape
    starts = (jnp.cumsum(record_len) - record_len).astype(jnp.int32)

    feat_final = pl.pallas_call(
        _dma_kernel,
        in_specs=[
            pl.BlockSpec(memory_space=pltpu.SMEM),
            pl.BlockSpec(memory_space=pltpu.MemorySpace.HBM),
            pl.BlockSpec(memory_space=pltpu.MemorySpace.HBM),
        ],
        out_specs=pl.BlockSpec(memory_space=pltpu.MemorySpace.HBM),
        out_shape=jax.ShapeDtypeStruct((B * H, C, Hh, W), feat_curr.dtype),
        scratch_shapes=[
            pltpu.SemaphoreType.DMA((B,)),
            pltpu.SemaphoreType.DMA((B,)),
        ],
    )(starts, feat_curr, feat_history)

    offset_loss = jnp.zeros((1,), dtype=feat_final.dtype)
    return (feat_final, offset_loss)


# speedup vs baseline: 41.8175x; 41.8175x over previous
import jax
import jax.numpy as jnp
from jax.experimental import pallas as pl
from jax.experimental.pallas import tpu as pltpu


def _copy_kernel(starts_ref, curr_ref, hist_ref, out_ref):
    del starts_ref
    k = pl.program_id(1)

    @pl.when(k == 0)
    def _():
        out_ref[...] = curr_ref[...]

    @pl.when(k != 0)
    def _():
        out_ref[...] = hist_ref[0]


def kernel(feat_curr, feat_history, record_len):
    B, H, C, Hh, W = feat_history.shape
    starts = (jnp.cumsum(record_len) - record_len).astype(jnp.int32)

    grid_spec = pltpu.PrefetchScalarGridSpec(
        num_scalar_prefetch=1,
        grid=(B, H),
        in_specs=[
            pl.BlockSpec((1, C, Hh, W), lambda bs, k, starts: (starts[bs], 0, 0, 0)),
            pl.BlockSpec(
                (1, 1, C, Hh, W),
                lambda bs, k, starts: (bs, jnp.maximum(k, 1), 0, 0, 0),
            ),
        ],
        out_specs=pl.BlockSpec(
            (1, C, Hh, W), lambda bs, k, starts: (bs * H + k, 0, 0, 0)
        ),
    )

    feat_final = pl.pallas_call(
        _copy_kernel,
        grid_spec=grid_spec,
        out_shape=jax.ShapeDtypeStruct((B * H, C, Hh, W), feat_curr.dtype),
    )(starts, feat_curr, feat_history)

    offset_loss = jnp.zeros((1,), dtype=feat_final.dtype)
    return (feat_final, offset_loss)
